# Initial kernel scaffold; baseline (speedup 1.0000x reference)
import jax
import jax.numpy as jnp
from jax.experimental import pallas as pl


def kernel(h, edge_index, edge_attr, mW1, mb1, mW2, mb2, mW3, mb3, mg, mbeta, uW1, ub1, uW2, ub2, uW3, ub3, ug, ubeta):
    raise NotImplementedError("write your pallas kernel here")


# speedup vs baseline: 3.8018x; 3.8018x over previous
import functools

import jax
import jax.numpy as jnp
from jax import lax
from jax.experimental import pallas as pl
from jax.experimental.pallas import tpu as pltpu
from jax.experimental.pallas import tpu_sc as plsc

N = 10000
E = 320000
D = 128
EA = 16

NC, NS, L = 2, 16, 16
NW = NC * NS
EPW = E // NW
CHUNK = 80
NCH = EPW // CHUNK
ROWS_PT = 624
ROWS_TAIL = N - NS * ROWS_PT

_MESH = plsc.VectorSubcoreMesh(core_axis_name="c", subcore_axis_name="s")


def _pq_body(h_ref, wa_ref, wb_ref, p_ref, q_ref):
    hb = h_ref[...]
    p_ref[...] = jnp.dot(hb, wa_ref[...], preferred_element_type=jnp.float32)
    q_ref[...] = jnp.dot(hb, wb_ref[...], preferred_element_type=jnp.float32)


def _compute_pq(h, wa, wb):
    nb = 10
    blk = N // nb
    return pl.pallas_call(
        _pq_body,
        grid=(nb,),
        in_specs=[
            pl.BlockSpec((blk, D), lambda i: (i, 0)),
            pl.BlockSpec((D, D), lambda i: (0, 0)),
            pl.BlockSpec((D, D), lambda i: (0, 0)),
        ],
        out_specs=[pl.BlockSpec((blk, D), lambda i: (i, 0))] * 2,
        out_shape=[jax.ShapeDtypeStruct((N, D), jnp.float32)] * 2,
    )(h, wa, wb)


@functools.partial(
    pl.kernel,
    out_type=jax.ShapeDtypeStruct((E, D), jnp.float32),
    mesh=_MESH,
    scratch_types=[
        pltpu.VMEM((NCH, CHUNK), jnp.int32),
        pltpu.VMEM((NCH, CHUNK), jnp.int32),
        pltpu.VMEM((CHUNK, D), jnp.float32),
        pltpu.VMEM((CHUNK, D), jnp.float32),
        pltpu.SemaphoreType.DMA,
        pltpu.SemaphoreType.DMA,
    ],
)
def _gather_add(p_hbm, q_hbm, src_hbm, dst_hbm, g_hbm,
                sidx, didx, pbuf, qbuf, sem_p, sem_q):
    c = lax.axis_index("c")
    s = lax.axis_index("s")
    wid = s * NC + c
    base = wid * EPW
    pltpu.sync_copy(src_hbm.at[wid], sidx)
    pltpu.sync_copy(dst_hbm.at[wid], didx)

    @pl.loop(0, NCH)
    def _chunk(j):
        cp = pltpu.async_copy(p_hbm.at[sidx.at[j]], pbuf, sem_p)
        cq = pltpu.async_copy(q_hbm.at[didx.at[j]], qbuf, sem_q)
        cp.wait()
        cq.wait()

        @pl.loop(0, CHUNK)
        def _row(r):
            for cc in range(D // L):
                sl = pl.ds(cc * L, L)
                plsc.addupdate(pbuf.at[r, sl], qbuf[r, sl])

        pltpu.sync_copy(pbuf, g_hbm.at[pl.ds(base + j * CHUNK, CHUNK)])


def _msg_body(g_ref, ea_ref, wc_ref, b1_ref, w2_ref, b2_ref, w3_ref, b3_ref,
              gg_ref, bb_ref, out_ref):
    x = g_ref[...] + jnp.dot(ea_ref[...], wc_ref[...],
                             preferred_element_type=jnp.float32) + b1_ref[...]
    x = jnp.maximum(x, 0.0)
    x = jnp.dot(x, w2_ref[...], preferred_element_type=jnp.float32) + b2_ref[...]
    x = jnp.maximum(x, 0.0)
    x = jnp.dot(x, w3_ref[...], preferred_element_type=jnp.float32) + b3_ref[...]
    mu = jnp.mean(x, axis=-1, keepdims=True)
    xc = x - mu
    var = jnp.mean(xc * xc, axis=-1, keepdims=True)
    out_ref[...] = xc * lax.rsqrt(var + 1e-5) * gg_ref[...] + bb_ref[...]


def _compute_msg(g, ea, wc, b1, w2, b2, w3, b3, gg, bb):
    be = 2560
    nb = E // be
    wspec = pl.BlockSpec((D, D), lambda i: (0, 0))
    vspec = pl.BlockSpec((1, D), lambda i: (0, 0))
    return pl.pallas_call(
        _msg_body,
        grid=(nb,),
        in_specs=[
            pl.BlockSpec((be, D), lambda i: (i, 0)),
            pl.BlockSpec((be, EA), lambda i: (i, 0)),
            pl.BlockSpec((EA, D), lambda i: (0, 0)),
            vspec, wspec, vspec, wspec, vspec, vspec, vspec,
        ],
        out_specs=pl.BlockSpec((be, D), lambda i: (i, 0)),
        out_shape=jax.ShapeDtypeStruct((E, D), jnp.float32),
    )(g, ea, wc, b1, w2, b2, w3, b3, gg, bb)


@functools.partial(
    pl.kernel,
    out_type=jax.ShapeDtypeStruct((NC, N, D), jnp.float32),
    mesh=_MESH,
    scratch_types=[
        pltpu.VMEM_SHARED((N, D), jnp.float32),
        pltpu.VMEM((NCH, CHUNK), jnp.int32),
        pltpu.VMEM((CHUNK, D), jnp.float32),
    ],
)
def _scatter_add(msg_hbm, dst_hbm, zeros_hbm, agg_hbm, acc, didx, mbuf):
    c = lax.axis_index("c")
    s = lax.axis_index("s")
    wid = s * NC + c
    pltpu.sync_copy(zeros_hbm.at[pl.ds(0, ROWS_PT)],
                    acc.at[pl.ds(s * ROWS_PT, ROWS_PT)])

    @pl.when(s == NS - 1)
    def _zero_tail():
        pltpu.sync_copy(zeros_hbm.at[pl.ds(0, ROWS_TAIL)],
                        acc.at[pl.ds(NS * ROWS_PT, ROWS_TAIL)])

    plsc.subcore_barrier()
    pltpu.sync_copy(dst_hbm.at[wid], didx)

    @pl.loop(0, NCH)
    def _chunk(j):
        pltpu.sync_copy(msg_hbm.at[pl.ds(wid * EPW + j * CHUNK, CHUNK)], mbuf)
        pltpu.sync_copy(mbuf, acc.at[didx.at[j]], add=True)

    plsc.subcore_barrier()
    pltpu.sync_copy(acc.at[pl.ds(s * ROWS_PT, ROWS_PT)],
                    agg_hbm.at[c, pl.ds(s * ROWS_PT, ROWS_PT)])

    @pl.when(s == NS - 1)
    def _drain_tail():
        pltpu.sync_copy(acc.at[pl.ds(NS * ROWS_PT, ROWS_TAIL)],
                        agg_hbm.at[c, pl.ds(NS * ROWS_PT, ROWS_TAIL)])


def _upd_body(h_ref, a0_ref, a1_ref, wa_ref, wb_ref, b1_ref, w2_ref, b2_ref,
              w3_ref, b3_ref, gg_ref, bb_ref, out_ref):
    hb = h_ref[...]
    agg = a0_ref[...] + a1_ref[...]
    x = (jnp.dot(hb, wa_ref[...], preferred_element_type=jnp.float32)
         + jnp.dot(agg, wb_ref[...], preferred_element_type=jnp.float32)
         + b1_ref[...])
    x = jnp.maximum(x, 0.0)
    x = jnp.dot(x, w2_ref[...], preferred_element_type=jnp.float32) + b2_ref[...]
    x = jnp.maximum(x, 0.0)
    x = jnp.dot(x, w3_ref[...], preferred_element_type=jnp.float32) + b3_ref[...]
    mu = jnp.mean(x, axis=-1, keepdims=True)
    xc = x - mu
    var = jnp.mean(xc * xc, axis=-1, keepdims=True)
    out_ref[...] = xc * lax.rsqrt(var + 1e-5) * gg_ref[...] + bb_ref[...] + hb


def _compute_update(h, a0, a1, wa, wb, b1, w2, b2, w3, b3, gg, bb):
    nb = 10
    blk = N // nb
    wspec = pl.BlockSpec((D, D), lambda i: (0, 0))
    vspec = pl.BlockSpec((1, D), lambda i: (0, 0))
    rspec = pl.BlockSpec((blk, D), lambda i: (i, 0))
    return pl.pallas_call(
        _upd_body,
        grid=(nb,),
        in_specs=[rspec, rspec, rspec, wspec, wspec, vspec, wspec, vspec,
                  wspec, vspec, vspec, vspec],
        out_specs=rspec,
        out_shape=jax.ShapeDtypeStruct((N, D), jnp.float32),
    )(h, a0, a1, wa, wb, b1, w2, b2, w3, b3, gg, bb)


def kernel(h, edge_index, edge_attr, mW1, mb1, mW2, mb2, mW3, mb3, mg, mbeta,
           uW1, ub1, uW2, ub2, uW3, ub3, ug, ubeta):
    src2d = edge_index[0].astype(jnp.int32).reshape(NW, NCH, CHUNK)
    dst2d = edge_index[1].astype(jnp.int32).reshape(NW, NCH, CHUNK)

    mA = mW1[:D]
    mB = mW1[D:2 * D]
    mC = mW1[2 * D:]
    uA = uW1[:D]
    uB = uW1[D:]

    mb1r = mb1.reshape(1, D)
    mb2r = mb2.reshape(1, D)
    mb3r = mb3.reshape(1, D)
    mgr = mg.reshape(1, D)
    mbetar = mbeta.reshape(1, D)
    ub1r = ub1.reshape(1, D)
    ub2r = ub2.reshape(1, D)
    ub3r = ub3.reshape(1, D)
    ugr = ug.reshape(1, D)
    ubetar = ubeta.reshape(1, D)

    p, q = _compute_pq(h, mA, mB)
    g = _gather_add(p, q, src2d, dst2d)
    msg = _compute_msg(g, edge_attr, mC, mb1r, mW2, mb2r, mW3, mb3r,
                       mgr, mbetar)
    zeros = jnp.zeros((ROWS_PT, D), dtype=jnp.float32)
    aggp = _scatter_add(msg, dst2d, zeros)
    h_new = _compute_update(h, aggp[0], aggp[1], uA, uB, ub1r, uW2, ub2r,
                            uW3, ub3r, ugr, ubetar)
    return (h_new, msg)


# speedup vs baseline: 4.2069x; 1.1065x over previous
import functools

import jax
import jax.numpy as jnp
from jax import lax
from jax.experimental import pallas as pl
from jax.experimental.pallas import tpu as pltpu
from jax.experimental.pallas import tpu_sc as plsc

N = 10000
E = 320000
D = 128
EA = 16

NC, NS, L = 2, 16, 16
NW = NC * NS
EPW = E // NW
CHUNK = 80
NCH = EPW // CHUNK
ROWS_PT = 624
ROWS_TAIL = N - NS * ROWS_PT

_MESH = plsc.VectorSubcoreMesh(core_axis_name="c", subcore_axis_name="s")


def _pq_body(h_ref, wa_ref, wb_ref, p_ref, q_ref):
    hb = h_ref[...]
    p_ref[...] = jnp.dot(hb, wa_ref[...], preferred_element_type=jnp.float32)
    q_ref[...] = jnp.dot(hb, wb_ref[...], preferred_element_type=jnp.float32)


def _compute_pq(h, wa, wb):
    nb = 10
    blk = N // nb
    return pl.pallas_call(
        _pq_body,
        grid=(nb,),
        in_specs=[
            pl.BlockSpec((blk, D), lambda i: (i, 0)),
            pl.BlockSpec((D, D), lambda i: (0, 0)),
            pl.BlockSpec((D, D), lambda i: (0, 0)),
        ],
        out_specs=[pl.BlockSpec((blk, D), lambda i: (i, 0))] * 2,
        out_shape=[jax.ShapeDtypeStruct((N, D), jnp.float32)] * 2,
    )(h, wa, wb)


@functools.partial(
    pl.kernel,
    out_type=jax.ShapeDtypeStruct((E, D), jnp.float32),
    mesh=_MESH,
    scratch_types=[
        pltpu.VMEM((NCH, CHUNK), jnp.int32),
        pltpu.VMEM((NCH, CHUNK), jnp.int32),
        pltpu.VMEM((2, CHUNK, D), jnp.float32),
        pltpu.VMEM((2, CHUNK, D), jnp.float32),
        pltpu.SemaphoreType.DMA,
        pltpu.SemaphoreType.DMA,
        pltpu.SemaphoreType.DMA,
        pltpu.SemaphoreType.DMA,
    ],
)
def _gather_add(p_hbm, q_hbm, src_hbm, dst_hbm, g_hbm,
                sidx, didx, pbuf, qbuf, sp0, sq0, sp1, sq1):
    c = lax.axis_index("c")
    s = lax.axis_index("s")
    wid = s * NC + c
    base = wid * EPW
    pltpu.sync_copy(src_hbm.at[wid], sidx)
    pltpu.sync_copy(dst_hbm.at[wid], didx)
    sems = ((sp0, sq0), (sp1, sq1))

    def _start(j, b):
        cp = pltpu.async_copy(p_hbm.at[sidx.at[j]], pbuf.at[b], sems[b][0])
        cq = pltpu.async_copy(q_hbm.at[didx.at[j]], qbuf.at[b], sems[b][1])
        return cp, cq

    def _finish(j, b, cp, cq):
        cp.wait()
        cq.wait()

        @pl.loop(0, CHUNK)
        def _row(r):
            for cc in range(D // L):
                sl = pl.ds(cc * L, L)
                plsc.addupdate(pbuf.at[b, r, sl], qbuf[b, r, sl])

        pltpu.sync_copy(pbuf.at[b], g_hbm.at[pl.ds(base + j * CHUNK, CHUNK)])

    @pl.loop(0, NCH - 1, step=2)
    def _chunk(j):
        cp0, cq0 = _start(j, 0)
        cp1, cq1 = _start(j + 1, 1)
        _finish(j, 0, cp0, cq0)
        _finish(j + 1, 1, cp1, cq1)

    cp, cq = _start(NCH - 1, 0)
    _finish(NCH - 1, 0, cp, cq)


def _msg_body(g_ref, ea_ref, wc_ref, b1_ref, w2_ref, b2_ref, w3_ref, b3_ref,
              gg_ref, bb_ref, out_ref):
    x = g_ref[...] + jnp.dot(ea_ref[...], wc_ref[...],
                             preferred_element_type=jnp.float32) + b1_ref[...]
    x = jnp.maximum(x, 0.0)
    x = jnp.dot(x, w2_ref[...], preferred_element_type=jnp.float32) + b2_ref[...]
    x = jnp.maximum(x, 0.0)
    x = jnp.dot(x, w3_ref[...], preferred_element_type=jnp.float32) + b3_ref[...]
    mu = jnp.mean(x, axis=-1, keepdims=True)
    xc = x - mu
    var = jnp.mean(xc * xc, axis=-1, keepdims=True)
    out_ref[...] = xc * lax.rsqrt(var + 1e-5) * gg_ref[...] + bb_ref[...]


def _compute_msg(g, ea, wc, b1, w2, b2, w3, b3, gg, bb):
    be = 2560
    nb = E // be
    wspec = pl.BlockSpec((D, D), lambda i: (0, 0))
    vspec = pl.BlockSpec((1, D), lambda i: (0, 0))
    return pl.pallas_call(
        _msg_body,
        grid=(nb,),
        in_specs=[
            pl.BlockSpec((be, D), lambda i: (i, 0)),
            pl.BlockSpec((be, EA), lambda i: (i, 0)),
            pl.BlockSpec((EA, D), lambda i: (0, 0)),
            vspec, wspec, vspec, wspec, vspec, vspec, vspec,
        ],
        out_specs=pl.BlockSpec((be, D), lambda i: (i, 0)),
        out_shape=jax.ShapeDtypeStruct((E, D), jnp.float32),
    )(g, ea, wc, b1, w2, b2, w3, b3, gg, bb)


@functools.partial(
    pl.kernel,
    out_type=jax.ShapeDtypeStruct((NC, N, D), jnp.float32),
    mesh=_MESH,
    scratch_types=[
        pltpu.VMEM_SHARED((N, D), jnp.float32),
        pltpu.VMEM((NCH, CHUNK), jnp.int32),
        pltpu.VMEM((2, CHUNK, D), jnp.float32),
        pltpu.SemaphoreType.DMA,
        pltpu.SemaphoreType.DMA,
    ],
)
def _scatter_add(msg_hbm, dst_hbm, zeros_hbm, agg_hbm, acc, didx, mbuf,
                 sm0, sm1):
    c = lax.axis_index("c")
    s = lax.axis_index("s")
    wid = s * NC + c
    pltpu.sync_copy(zeros_hbm.at[pl.ds(0, ROWS_PT)],
                    acc.at[pl.ds(s * ROWS_PT, ROWS_PT)])

    @pl.when(s == NS - 1)
    def _zero_tail():
        pltpu.sync_copy(zeros_hbm.at[pl.ds(0, ROWS_TAIL)],
                        acc.at[pl.ds(NS * ROWS_PT, ROWS_TAIL)])

    plsc.subcore_barrier()
    pltpu.sync_copy(dst_hbm.at[wid], didx)

    base = wid * EPW
    sems = (sm0, sm1)

    def _start(j, b):
        return pltpu.async_copy(
            msg_hbm.at[pl.ds(base + j * CHUNK, CHUNK)], mbuf.at[b], sems[b])

    def _finish(j, b, cm):
        cm.wait()
        pltpu.sync_copy(mbuf.at[b], acc.at[didx.at[j]], add=True)

    @pl.loop(0, NCH - 1, step=2)
    def _chunk(j):
        cm0 = _start(j, 0)
        cm1 = _start(j + 1, 1)
        _finish(j, 0, cm0)
        _finish(j + 1, 1, cm1)

    cm = _start(NCH - 1, 0)
    _finish(NCH - 1, 0, cm)

    plsc.subcore_barrier()
    pltpu.sync_copy(acc.at[pl.ds(s * ROWS_PT, ROWS_PT)],
                    agg_hbm.at[c, pl.ds(s * ROWS_PT, ROWS_PT)])

    @pl.when(s == NS - 1)
    def _drain_tail():
        pltpu.sync_copy(acc.at[pl.ds(NS * ROWS_PT, ROWS_TAIL)],
                        agg_hbm.at[c, pl.ds(NS * ROWS_PT, ROWS_TAIL)])


def _upd_body(h_ref, a0_ref, a1_ref, wa_ref, wb_ref, b1_ref, w2_ref, b2_ref,
              w3_ref, b3_ref, gg_ref, bb_ref, out_ref):
    hb = h_ref[...]
    agg = a0_ref[...] + a1_ref[...]
    x = (jnp.dot(hb, wa_ref[...], preferred_element_type=jnp.float32)
         + jnp.dot(agg, wb_ref[...], preferred_element_type=jnp.float32)
         + b1_ref[...])
    x = jnp.maximum(x, 0.0)
    x = jnp.dot(x, w2_ref[...], preferred_element_type=jnp.float32) + b2_ref[...]
    x = jnp.maximum(x, 0.0)
    x = jnp.dot(x, w3_ref[...], preferred_element_type=jnp.float32) + b3_ref[...]
    mu = jnp.mean(x, axis=-1, keepdims=True)
    xc = x - mu
    var = jnp.mean(xc * xc, axis=-1, keepdims=True)
    out_ref[...] = xc * lax.rsqrt(var + 1e-5) * gg_ref[...] + bb_ref[...] + hb


def _compute_update(h, a0, a1, wa, wb, b1, w2, b2, w3, b3, gg, bb):
    nb = 10
    blk = N // nb
    wspec = pl.BlockSpec((D, D), lambda i: (0, 0))
    vspec = pl.BlockSpec((1, D), lambda i: (0, 0))
    rspec = pl.BlockSpec((blk, D), lambda i: (i, 0))
    return pl.pallas_call(
        _upd_body,
        grid=(nb,),
        in_specs=[rspec, rspec, rspec, wspec, wspec, vspec, wspec, vspec,
                  wspec, vspec, vspec, vspec],
        out_specs=rspec,
        out_shape=jax.ShapeDtypeStruct((N, D), jnp.float32),
    )(h, a0, a1, wa, wb, b1, w2, b2, w3, b3, gg, bb)


def kernel(h, edge_index, edge_attr, mW1, mb1, mW2, mb2, mW3, mb3, mg, mbeta,
           uW1, ub1, uW2, ub2, uW3, ub3, ug, ubeta):
    src2d = edge_index[0].astype(jnp.int32).reshape(NW, NCH, CHUNK)
    dst2d = edge_index[1].astype(jnp.int32).reshape(NW, NCH, CHUNK)

    mA = mW1[:D]
    mB = mW1[D:2 * D]
    mC = mW1[2 * D:]
    uA = uW1[:D]
    uB = uW1[D:]

    mb1r = mb1.reshape(1, D)
    mb2r = mb2.reshape(1, D)
    mb3r = mb3.reshape(1, D)
    mgr = mg.reshape(1, D)
    mbetar = mbeta.reshape(1, D)
    ub1r = ub1.reshape(1, D)
    ub2r = ub2.reshape(1, D)
    ub3r = ub3.reshape(1, D)
    ugr = ug.reshape(1, D)
    ubetar = ubeta.reshape(1, D)

    p, q = _compute_pq(h, mA, mB)
    g = _gather_add(p, q, src2d, dst2d)
    msg = _compute_msg(g, edge_attr, mC, mb1r, mW2, mb2r, mW3, mb3r,
                       mgr, mbetar)
    zeros = jnp.zeros((ROWS_PT, D), dtype=jnp.float32)
    aggp = _scatter_add(msg, dst2d, zeros)
    h_new = _compute_update(h, aggp[0], aggp[1], uA, uB, ub1r, uW2, ub2r,
                            uW3, ub3r, ugr, ubetar)
    return (h_new, msg)
